# Initial kernel scaffold; baseline (speedup 1.0000x reference)
import jax
import jax.numpy as jnp
from jax.experimental import pallas as pl


def kernel(x, neigh_orders, g1, b1, g2, b2, g3, b3, g4, b4, W1, wb1, W2, wb2, W3, wb3, W4, wb4):
    raise NotImplementedError("write your pallas kernel here")


# speedup vs baseline: 1.1543x; 1.1543x over previous
import functools

import jax
import jax.numpy as jnp
from jax.experimental import pallas as pl
from jax.experimental.pallas import tpu as pltpu

N = 163842
CH = 64
EPS = 1e-5
SLOPE = 0.2

GB = 512
NB2 = 161
NBLK = 2 * NB2
NP = NBLK * GB
NPH = NP // 2
SB = 1024
U = 32


def _stats_kernel(x_ref, o_ref, acc_ref):
    i = pl.program_id(0)

    @pl.when(i == 0)
    def _():
        acc_ref[...] = jnp.zeros_like(acc_ref)

    xb = x_ref[...]
    acc_ref[0:1, :] = acc_ref[0:1, :] + jnp.sum(xb, axis=0, keepdims=True)
    acc_ref[1:2, :] = acc_ref[1:2, :] + jnp.sum(xb * xb, axis=0, keepdims=True)

    @pl.when(i == pl.num_programs(0) - 1)
    def _():
        mu = acc_ref[0:1, :] * (1.0 / N)
        var = acc_ref[1:2, :] * (1.0 / N) - mu * mu
        o_ref[0:1, :] = mu
        o_ref[1:2, :] = var


def _stats(xp):
    return pl.pallas_call(
        _stats_kernel,
        grid=(NP // SB,),
        in_specs=[pl.BlockSpec((SB, CH), lambda i: (i, 0))],
        out_specs=pl.BlockSpec((2, CH), lambda i: (0, 0)),
        out_shape=jax.ShapeDtypeStruct((2, CH), jnp.float32),
        scratch_shapes=[pltpu.VMEM((2, CH), jnp.float32)],
        compiler_params=pltpu.CompilerParams(
            dimension_semantics=("arbitrary",)),
        name="bn_stats",
    )(xp)


def _transform_kernel(k, *refs):
    x_refs = refs[:k]
    mu_ref, var_ref, g_ref, b_ref, w2_ref, wb_ref = refs[k:k + 6]
    z_refs = refs[k + 6:k + 13]
    y_ref = refs[k + 13]
    for m in range(k):
        sl = slice(m * CH, (m + 1) * CH)
        xm = x_refs[m][...]
        yv = ((xm - mu_ref[0:1, sl])
              * jax.lax.rsqrt(var_ref[0:1, sl] + EPS)
              * g_ref[0:1, sl] + b_ref[0:1, sl])
        y_ref[:, sl] = jnp.where(yv > 0, yv, SLOPE * yv)
    z = jnp.dot(y_ref[...], w2_ref[...],
                preferred_element_type=jnp.float32) + wb_ref[...]
    for j in range(7):
        z_refs[j][...] = z[:, j * CH:(j + 1) * CH]


def _transform(k, xs, mu, var, g, b, w2, wbpad):
    cin = CH * k
    row_spec = pl.BlockSpec((GB, CH), lambda i: (i, 0))

    def cspec(shape):
        return pl.BlockSpec(shape, lambda i: tuple(0 for _ in shape))

    return pl.pallas_call(
        functools.partial(_transform_kernel, k),
        grid=(NBLK,),
        in_specs=[row_spec] * k + [cspec((1, cin)), cspec((1, cin)),
                                   cspec((1, cin)), cspec((1, cin)),
                                   cspec((cin, 7 * CH)), cspec((1, 7 * CH))],
        out_specs=[pl.BlockSpec((GB, CH), lambda i: (i, 0))] * 7,
        out_shape=[jax.ShapeDtypeStruct((NP, CH), jnp.float32)] * 7,
        scratch_shapes=[pltpu.VMEM((GB, cin), jnp.float32)],
        compiler_params=pltpu.CompilerParams(
            dimension_semantics=("arbitrary",)),
        name=f"transform{k}",
    )(*xs, mu, var, g, b, w2, wbpad)


def _gather_kernel(first_j, last_j, *refs):
    if first_j:
        tab_hbm, idx_ref, o_ref, tab_vmem, sem = refs
        prev_ref = None
    else:
        tab_hbm, idx_ref, prev_ref, o_ref, tab_vmem, sem = refs
    i = pl.program_id(0)

    @pl.when(i == 0)
    def _():
        pltpu.make_async_copy(tab_hbm, tab_vmem, sem).start()
        pltpu.make_async_copy(tab_hbm, tab_vmem, sem).wait()

    def outer(o, carry):
        base = o * U
        for u in range(U):
            mi = base + u
            raw = idx_ref[0, 0, mi]
            r = jax.lax.shift_right_logical(raw, 1)
            h = jnp.bitwise_and(raw, 1)
            row = tab_vmem[r]
            sel = jnp.where(h == 0, row[:, :CH], row[:, CH:])
            o_ref[pl.ds(mi, 1)] = sel.reshape(1, 1, CH)
        return carry

    jax.lax.fori_loop(0, GB // U, outer, 0)

    if not first_j:
        o_ref[...] = o_ref[...] + prev_ref[...]
    if last_j:
        base_row = i * GB
        rows = base_row + jax.lax.broadcasted_iota(jnp.int32, (GB, 1, CH), 0)
        o_ref[...] = jnp.where(rows < N, o_ref[...], 0.0)


def _gather(first_j, last_j, tab, idx_col, prev):
    row_spec = pl.BlockSpec((GB, 1, CH), lambda i: (i, 0, 0))
    in_specs = [
        pl.BlockSpec(memory_space=pl.ANY),
        pl.BlockSpec((1, 1, GB), lambda i: (i, 0, 0),
                     memory_space=pltpu.SMEM),
    ]
    args = [tab, idx_col]
    aliases = {}
    if not first_j:
        in_specs.append(row_spec)
        args.append(prev)
        aliases = {2: 0}
    return pl.pallas_call(
        functools.partial(_gather_kernel, first_j, last_j),
        grid=(NBLK,),
        in_specs=in_specs,
        out_specs=row_spec,
        out_shape=jax.ShapeDtypeStruct((NP, 1, CH), jnp.float32),
        scratch_shapes=[pltpu.VMEM((NPH, 1, 128), jnp.float32),
                        pltpu.SemaphoreType.DMA],
        input_output_aliases=aliases,
        compiler_params=pltpu.CompilerParams(
            dimension_semantics=("arbitrary",),
            vmem_limit_bytes=48 * 1024 * 1024),
        name="gather_acc",
    )(*args)


def kernel(x, neigh_orders, g1, b1, g2, b2, g3, b3, g4, b4,
           W1, wb1, W2, wb2, W3, wb3, W4, wb4):
    xp = jnp.pad(x, ((0, NP - N), (0, 0)))
    idx = neigh_orders.astype(jnp.int32).reshape(N, 7)
    idxp = jnp.pad(idx, ((0, NP - N), (0, 0)))
    idx_cols = [idxp[:, j].reshape(NBLK, 1, GB) for j in range(7)]

    gb = [(g1, b1), (g2, b2), (g3, b3), (g4, b4)]
    ww = [(W1, wb1), (W2, wb2), (W3, wb3), (W4, wb4)]

    blocks = [xp]
    stats = []
    for k in range(1, 5):
        cin = CH * k
        stats.append(_stats(blocks[-1]))
        mu = jnp.concatenate([s[0:1] for s in stats], axis=1)
        var = jnp.concatenate([s[1:2] for s in stats], axis=1)
        g, b = gb[k - 1]
        W, wb = ww[k - 1]
        w2 = W.reshape(7, cin, CH).transpose(1, 0, 2).reshape(cin, 7 * CH)
        wbpad = jnp.concatenate(
            [wb.reshape(1, CH), jnp.zeros((1, 6 * CH), jnp.float32)], axis=1)
        zs = _transform(k, blocks, mu, var, g.reshape(1, cin),
                        b.reshape(1, cin), w2, wbpad)
        out = None
        for j in range(7):
            tab = zs[j].reshape(NPH, 1, 128)
            out = _gather(j == 0, j == 6, tab, idx_cols[j], out)
        blocks.append(out.reshape(NP, CH))
    return blocks[4][:N]


# speedup vs baseline: 1.2319x; 1.0672x over previous
import functools

import jax
import jax.numpy as jnp
from jax.experimental import pallas as pl
from jax.experimental.pallas import tpu as pltpu

N = 163842
CH = 64
EPS = 1e-5
SLOPE = 0.2

GB = 512
NB2 = 161
NBLK = 2 * NB2
NP = NBLK * GB
NPH = NP // 2
SB = 1024
U = 32


def _stats_kernel(x_ref, o_ref, acc_ref):
    i = pl.program_id(0)

    @pl.when(i == 0)
    def _():
        acc_ref[...] = jnp.zeros_like(acc_ref)

    xb = x_ref[...]
    acc_ref[0:1, :] = acc_ref[0:1, :] + jnp.sum(xb, axis=0, keepdims=True)
    acc_ref[1:2, :] = acc_ref[1:2, :] + jnp.sum(xb * xb, axis=0, keepdims=True)

    @pl.when(i == pl.num_programs(0) - 1)
    def _():
        mu = acc_ref[0:1, :] * (1.0 / N)
        var = acc_ref[1:2, :] * (1.0 / N) - mu * mu
        o_ref[0:1, :] = mu
        o_ref[1:2, :] = var


def _stats(xp):
    return pl.pallas_call(
        _stats_kernel,
        grid=(NP // SB,),
        in_specs=[pl.BlockSpec((SB, CH), lambda i: (i, 0))],
        out_specs=pl.BlockSpec((2, CH), lambda i: (0, 0)),
        out_shape=jax.ShapeDtypeStruct((2, CH), jnp.float32),
        scratch_shapes=[pltpu.VMEM((2, CH), jnp.float32)],
        compiler_params=pltpu.CompilerParams(
            dimension_semantics=("arbitrary",)),
        name="bn_stats",
    )(xp)


def _transform_kernel(k, *refs):
    x_refs = refs[:k]
    mu_ref, var_ref, g_ref, b_ref, w2_ref, wb_ref = refs[k:k + 6]
    z_refs = refs[k + 6:k + 13]
    y_ref = refs[k + 13]
    for m in range(k):
        sl = slice(m * CH, (m + 1) * CH)
        xm = x_refs[m][...]
        yv = ((xm - mu_ref[0:1, sl])
              * jax.lax.rsqrt(var_ref[0:1, sl] + EPS)
              * g_ref[0:1, sl] + b_ref[0:1, sl])
        y_ref[:, sl] = jnp.where(yv > 0, yv, SLOPE * yv)
    z = jnp.dot(y_ref[...], w2_ref[...],
                preferred_element_type=jnp.float32) + wb_ref[...]
    for j in range(7):
        z_refs[j][...] = z[:, j * CH:(j + 1) * CH]


def _transform(k, xs, mu, var, g, b, w2, wbpad):
    cin = CH * k
    row_spec = pl.BlockSpec((GB, CH), lambda i: (i, 0))

    def cspec(shape):
        return pl.BlockSpec(shape, lambda i: tuple(0 for _ in shape))

    return pl.pallas_call(
        functools.partial(_transform_kernel, k),
        grid=(NBLK,),
        in_specs=[row_spec] * k + [cspec((1, cin)), cspec((1, cin)),
                                   cspec((1, cin)), cspec((1, cin)),
                                   cspec((cin, 7 * CH)), cspec((1, 7 * CH))],
        out_specs=[pl.BlockSpec((GB, CH), lambda i: (i, 0))] * 7,
        out_shape=[jax.ShapeDtypeStruct((NP, CH), jnp.float32)] * 7,
        scratch_shapes=[pltpu.VMEM((GB, cin), jnp.float32)],
        compiler_params=pltpu.CompilerParams(
            dimension_semantics=("arbitrary",)),
        name=f"transform{k}",
    )(*xs, mu, var, g, b, w2, wbpad)


def _gather_kernel(first_j, last_j, *refs):
    if first_j:
        tab_hbm, ridx_ref, hm_ref, o_ref, tab_vmem, gt_ref, sem = refs
        prev_ref = None
    else:
        tab_hbm, ridx_ref, hm_ref, prev_ref, o_ref, tab_vmem, gt_ref, sem = refs
    i = pl.program_id(0)

    @pl.when(i == 0)
    def _():
        pltpu.make_async_copy(tab_hbm, tab_vmem, sem).start()
        pltpu.make_async_copy(tab_hbm, tab_vmem, sem).wait()

    def outer(o, carry):
        base = o * U
        for u in range(U):
            mi = base + u
            gt_ref[pl.ds(mi, 1)] = tab_vmem[pl.ds(ridx_ref[0, 0, mi], 1)]
        return carry

    jax.lax.fori_loop(0, GB // U, outer, 0)

    g = gt_ref[...]
    lo = g[:, :, :CH]
    hi = g[:, :, CH:]
    hm = hm_ref[...]
    sel = lo + hm * (hi - lo)
    if not first_j:
        sel = sel + prev_ref[...]
    if last_j:
        rows = i * GB + jax.lax.broadcasted_iota(jnp.int32, (GB, 1, CH), 0)
        sel = jnp.where(rows < N, sel, 0.0)
    o_ref[...] = sel


def _gather(first_j, last_j, tab, ridx_col, hmask, prev):
    row_spec = pl.BlockSpec((GB, 1, CH), lambda i: (i, 0, 0))
    in_specs = [
        pl.BlockSpec(memory_space=pl.ANY),
        pl.BlockSpec((1, 1, GB), lambda i: (i, 0, 0),
                     memory_space=pltpu.SMEM),
        row_spec,
    ]
    args = [tab, ridx_col, hmask]
    aliases = {}
    if not first_j:
        in_specs.append(row_spec)
        args.append(prev)
        aliases = {3: 0}
    return pl.pallas_call(
        functools.partial(_gather_kernel, first_j, last_j),
        grid=(NBLK,),
        in_specs=in_specs,
        out_specs=row_spec,
        out_shape=jax.ShapeDtypeStruct((NP, 1, CH), jnp.float32),
        scratch_shapes=[pltpu.VMEM((NPH, 1, 128), jnp.float32),
                        pltpu.VMEM((GB, 1, 128), jnp.float32),
                        pltpu.SemaphoreType.DMA],
        input_output_aliases=aliases,
        compiler_params=pltpu.CompilerParams(
            dimension_semantics=("arbitrary",),
            vmem_limit_bytes=48 * 1024 * 1024),
        name="gather_acc",
    )(*args)


def kernel(x, neigh_orders, g1, b1, g2, b2, g3, b3, g4, b4,
           W1, wb1, W2, wb2, W3, wb3, W4, wb4):
    xp = jnp.pad(x, ((0, NP - N), (0, 0)))
    idx = neigh_orders.astype(jnp.int32).reshape(N, 7)
    idxp = jnp.pad(idx, ((0, NP - N), (0, 0)))
    ridx_cols = [(idxp[:, j] >> 1).reshape(NBLK, 1, GB) for j in range(7)]
    hmasks = [jnp.broadcast_to(
        (idxp[:, j] & 1).astype(jnp.float32).reshape(NP, 1, 1), (NP, 1, CH))
        for j in range(7)]

    gb = [(g1, b1), (g2, b2), (g3, b3), (g4, b4)]
    ww = [(W1, wb1), (W2, wb2), (W3, wb3), (W4, wb4)]

    blocks = [xp]
    stats = []
    for k in range(1, 5):
        cin = CH * k
        stats.append(_stats(blocks[-1]))
        mu = jnp.concatenate([s[0:1] for s in stats], axis=1)
        var = jnp.concatenate([s[1:2] for s in stats], axis=1)
        g, b = gb[k - 1]
        W, wb = ww[k - 1]
        w2 = W.reshape(7, cin, CH).transpose(1, 0, 2).reshape(cin, 7 * CH)
        wbpad = jnp.concatenate(
            [wb.reshape(1, CH), jnp.zeros((1, 6 * CH), jnp.float32)], axis=1)
        zs = _transform(k, blocks, mu, var, g.reshape(1, cin),
                        b.reshape(1, cin), w2, wbpad)
        out = None
        for j in range(7):
            tab = zs[j].reshape(NPH, 1, 128)
            out = _gather(j == 0, j == 6, tab, ridx_cols[j], hmasks[j], out)
        blocks.append(out.reshape(NP, CH))
    return blocks[4][:N]


# speedup vs baseline: 1.6391x; 1.3305x over previous
import functools

import jax
import jax.numpy as jnp
from jax.experimental import pallas as pl
from jax.experimental.pallas import tpu as pltpu

N = 163842
CH = 64
EPS = 1e-5
SLOPE = 0.2

GB = 1024
NBLK = 161
NP = NBLK * GB
NPH = NP // 2
SB = 1024
U = 64


def _fold(blk):
    return blk[:, :CH] + blk[:, CH:]


def _stats_kernel(x_ref, o_ref, acc_ref):
    i = pl.program_id(0)

    @pl.when(i == 0)
    def _():
        acc_ref[...] = jnp.zeros_like(acc_ref)

    xb = _fold(x_ref[...])
    rows = i * SB + jax.lax.broadcasted_iota(jnp.int32, xb.shape, 0)
    xb = jnp.where(rows < N, xb, 0.0)
    acc_ref[0:1, :] = acc_ref[0:1, :] + jnp.sum(xb, axis=0, keepdims=True)
    acc_ref[1:2, :] = acc_ref[1:2, :] + jnp.sum(xb * xb, axis=0, keepdims=True)

    @pl.when(i == pl.num_programs(0) - 1)
    def _():
        mu = acc_ref[0:1, :] * (1.0 / N)
        var = acc_ref[1:2, :] * (1.0 / N) - mu * mu
        o_ref[0:1, :] = mu
        o_ref[1:2, :] = var


def _stats(xp128):
    return pl.pallas_call(
        _stats_kernel,
        grid=(NP // SB,),
        in_specs=[pl.BlockSpec((SB, 128), lambda i: (i, 0))],
        out_specs=pl.BlockSpec((2, CH), lambda i: (0, 0)),
        out_shape=jax.ShapeDtypeStruct((2, CH), jnp.float32),
        scratch_shapes=[pltpu.VMEM((2, CH), jnp.float32)],
        compiler_params=pltpu.CompilerParams(
            dimension_semantics=("arbitrary",)),
        name="bn_stats",
    )(xp128)


def _transform_kernel(k, *refs):
    x_refs = refs[:k]
    mu_ref, var_ref, g_ref, b_ref, w2_ref, wb_ref = refs[k:k + 6]
    z_refs = refs[k + 6:k + 13]
    y_ref = refs[k + 13]
    for m in range(k):
        sl = slice(m * CH, (m + 1) * CH)
        xm = _fold(x_refs[m][...])
        yv = ((xm - mu_ref[0:1, sl])
              * jax.lax.rsqrt(var_ref[0:1, sl] + EPS)
              * g_ref[0:1, sl] + b_ref[0:1, sl])
        y_ref[:, sl] = jnp.where(yv > 0, yv, SLOPE * yv)
    z = jnp.dot(y_ref[...], w2_ref[...],
                preferred_element_type=jnp.float32) + wb_ref[...]
    for j in range(7):
        z_refs[j][...] = z[:, j * CH:(j + 1) * CH]


def _transform(k, xs, mu, var, g, b, w2, wbpad):
    cin = CH * k
    row_spec = pl.BlockSpec((GB, 128), lambda i: (i, 0))

    def cspec(shape):
        return pl.BlockSpec(shape, lambda i: tuple(0 for _ in shape))

    return pl.pallas_call(
        functools.partial(_transform_kernel, k),
        grid=(NBLK,),
        in_specs=[row_spec] * k + [cspec((1, cin)), cspec((1, cin)),
                                   cspec((1, cin)), cspec((1, cin)),
                                   cspec((cin, 7 * CH)), cspec((1, 7 * CH))],
        out_specs=[pl.BlockSpec((GB, CH), lambda i: (i, 0))] * 7,
        out_shape=[jax.ShapeDtypeStruct((NP, CH), jnp.float32)] * 7,
        scratch_shapes=[pltpu.VMEM((GB, cin), jnp.float32)],
        compiler_params=pltpu.CompilerParams(
            dimension_semantics=("arbitrary",)),
        name=f"transform{k}",
    )(*xs, mu, var, g, b, w2, wbpad)


def _gather_kernel(first_j, *refs):
    if first_j:
        tab_hbm, ridx_ref, hm_ref, o_ref, tab_vmem, gt_ref, sem = refs
        prev_ref = None
    else:
        tab_hbm, ridx_ref, hm_ref, prev_ref, o_ref, tab_vmem, gt_ref, sem = refs
    i = pl.program_id(0)

    @pl.when(i == 0)
    def _():
        pltpu.make_async_copy(tab_hbm, tab_vmem, sem).start()
        pltpu.make_async_copy(tab_hbm, tab_vmem, sem).wait()

    def outer(o, carry):
        base = o * U
        for u in range(U):
            mi = base + u
            gt_ref[pl.ds(mi, 1)] = tab_vmem[pl.ds(ridx_ref[0, 0, mi], 1)]
        return carry

    jax.lax.fori_loop(0, GB // U, outer, 0)

    sel = gt_ref[...] * hm_ref[...]
    if not first_j:
        sel = sel + prev_ref[...]
    o_ref[...] = sel


def _gather(first_j, tab, ridx_col, hmask, prev):
    row_spec = pl.BlockSpec((GB, 1, 128), lambda i: (i, 0, 0))
    in_specs = [
        pl.BlockSpec(memory_space=pl.ANY),
        pl.BlockSpec((1, 1, GB), lambda i: (i, 0, 0),
                     memory_space=pltpu.SMEM),
        row_spec,
    ]
    args = [tab, ridx_col, hmask]
    aliases = {}
    if not first_j:
        in_specs.append(row_spec)
        args.append(prev)
        aliases = {3: 0}
    return pl.pallas_call(
        functools.partial(_gather_kernel, first_j),
        grid=(NBLK,),
        in_specs=in_specs,
        out_specs=row_spec,
        out_shape=jax.ShapeDtypeStruct((NP, 1, 128), jnp.float32),
        scratch_shapes=[pltpu.VMEM((NPH, 1, 128), jnp.float32),
                        pltpu.VMEM((GB, 1, 128), jnp.float32),
                        pltpu.SemaphoreType.DMA],
        input_output_aliases=aliases,
        compiler_params=pltpu.CompilerParams(
            dimension_semantics=("arbitrary",),
            vmem_limit_bytes=50 * 1024 * 1024),
        name="gather_acc",
    )(*args)


def _fold_final_kernel(x_ref, o_ref):
    o_ref[...] = _fold(x_ref[...])


def _fold_final(xk128):
    return pl.pallas_call(
        _fold_final_kernel,
        grid=(NP // SB,),
        in_specs=[pl.BlockSpec((SB, 128), lambda i: (i, 0))],
        out_specs=pl.BlockSpec((SB, CH), lambda i: (i, 0)),
        out_shape=jax.ShapeDtypeStruct((NP, CH), jnp.float32),
        compiler_params=pltpu.CompilerParams(
            dimension_semantics=("arbitrary",)),
        name="fold_out",
    )(xk128)


def kernel(x, neigh_orders, g1, b1, g2, b2, g3, b3, g4, b4,
           W1, wb1, W2, wb2, W3, wb3, W4, wb4):
    xp = jnp.pad(x, ((0, NP - N), (0, 0)))
    xp128 = jnp.concatenate([xp, jnp.zeros_like(xp)], axis=1)
    idx = neigh_orders.astype(jnp.int32).reshape(N, 7)
    idxp = jnp.pad(idx, ((0, NP - N), (0, 0)))
    ridx_cols = [(idxp[:, j] >> 1).reshape(NBLK, 1, GB) for j in range(7)]
    hmasks = []
    for j in range(7):
        hj = (idxp[:, j] & 1).astype(jnp.float32).reshape(NP, 1, 1)
        hmasks.append(jnp.concatenate(
            [jnp.broadcast_to(1.0 - hj, (NP, 1, CH)),
             jnp.broadcast_to(hj, (NP, 1, CH))], axis=2))

    gb = [(g1, b1), (g2, b2), (g3, b3), (g4, b4)]
    ww = [(W1, wb1), (W2, wb2), (W3, wb3), (W4, wb4)]

    blocks = [xp128]
    stats = []
    for k in range(1, 5):
        cin = CH * k
        stats.append(_stats(blocks[-1]))
        mu = jnp.concatenate([s[0:1] for s in stats], axis=1)
        var = jnp.concatenate([s[1:2] for s in stats], axis=1)
        g, b = gb[k - 1]
        W, wb = ww[k - 1]
        w2 = W.reshape(7, cin, CH).transpose(1, 0, 2).reshape(cin, 7 * CH)
        wbpad = jnp.concatenate(
            [wb.reshape(1, CH), jnp.zeros((1, 6 * CH), jnp.float32)], axis=1)
        zs = _transform(k, blocks, mu, var, g.reshape(1, cin),
                        b.reshape(1, cin), w2, wbpad)
        out = None
        for j in range(7):
            tab = zs[j].reshape(NPH, 1, 128)
            out = _gather(j == 0, tab, ridx_cols[j], hmasks[j], out)
        blocks.append(out.reshape(NP, 128))
    return _fold_final(blocks[4])[:N]


# speedup vs baseline: 2.6627x; 1.6245x over previous
import functools

import jax
import jax.numpy as jnp
from jax.experimental import pallas as pl
from jax.experimental.pallas import tpu as pltpu

N = 163842
CH = 64
EPS = 1e-5
SLOPE = 0.2

GB = 1024
NBLK = 161
NP = NBLK * GB
NPH = NP // 2
SB = 7168
U = 64


def _stats_kernel(x_ref, o_ref, acc_ref):
    i = pl.program_id(0)

    @pl.when(i == 0)
    def _():
        acc_ref[...] = jnp.zeros_like(acc_ref)

    xb = x_ref[...]
    acc_ref[0:1, :] = acc_ref[0:1, :] + jnp.sum(xb, axis=0, keepdims=True)
    acc_ref[1:2, :] = acc_ref[1:2, :] + jnp.sum(xb * xb, axis=0, keepdims=True)

    @pl.when(i == pl.num_programs(0) - 1)
    def _():
        mu = acc_ref[0:1, :] * (1.0 / N)
        var = acc_ref[1:2, :] * (1.0 / N) - mu * mu
        o_ref[0:1, :] = mu
        o_ref[1:2, :] = var


def _stats(xp):
    return pl.pallas_call(
        _stats_kernel,
        grid=(NP // SB,),
        in_specs=[pl.BlockSpec((SB, CH), lambda i: (i, 0))],
        out_specs=pl.BlockSpec((2, CH), lambda i: (0, 0)),
        out_shape=jax.ShapeDtypeStruct((2, CH), jnp.float32),
        scratch_shapes=[pltpu.VMEM((2, CH), jnp.float32)],
        compiler_params=pltpu.CompilerParams(
            dimension_semantics=("arbitrary",)),
        name="bn_stats",
    )(xp)


def _transform_kernel(k, *refs):
    x_refs = refs[:k]
    mu_ref, var_ref, g_ref, b_ref, w2_ref, wb_ref = refs[k:k + 6]
    z_refs = refs[k + 6:k + 13]
    y_ref = refs[k + 13]
    for m in range(k):
        sl = slice(m * CH, (m + 1) * CH)
        xm = x_refs[m][...]
        yv = ((xm - mu_ref[0:1, sl])
              * jax.lax.rsqrt(var_ref[0:1, sl] + EPS)
              * g_ref[0:1, sl] + b_ref[0:1, sl])
        y_ref[:, sl] = jnp.where(yv > 0, yv, SLOPE * yv)
    z = jnp.dot(y_ref[...], w2_ref[...],
                preferred_element_type=jnp.float32) + wb_ref[...]
    z3 = z.reshape(GB // 2, 2, 7 * CH)
    for j in range(7):
        sl = slice(j * CH, (j + 1) * CH)
        packed = jnp.concatenate([z3[:, 0, sl], z3[:, 1, sl]], axis=-1)
        z_refs[j][...] = packed.reshape(GB // 2, 1, 128)


def _transform(k, xs, mu, var, g, b, w2, wbpad):
    cin = CH * k
    row_spec = pl.BlockSpec((GB, CH), lambda i: (i, 0))

    def cspec(shape):
        return pl.BlockSpec(shape, lambda i: tuple(0 for _ in shape))

    return pl.pallas_call(
        functools.partial(_transform_kernel, k),
        grid=(NBLK,),
        in_specs=[row_spec] * k + [cspec((1, cin)), cspec((1, cin)),
                                   cspec((1, cin)), cspec((1, cin)),
                                   cspec((cin, 7 * CH)), cspec((1, 7 * CH))],
        out_specs=[pl.BlockSpec((GB // 2, 1, 128), lambda i: (i, 0, 0))] * 7,
        out_shape=[jax.ShapeDtypeStruct((NPH, 1, 128), jnp.float32)] * 7,
        scratch_shapes=[pltpu.VMEM((GB, cin), jnp.float32)],
        compiler_params=pltpu.CompilerParams(
            dimension_semantics=("arbitrary",)),
        name=f"transform{k}",
    )(*xs, mu, var, g, b, w2, wbpad)


def _gather_kernel(tab_hbm, ridx_ref, o_ref, tab_vmem, sem):
    i = pl.program_id(0)

    @pl.when(i == 0)
    def _():
        pltpu.make_async_copy(tab_hbm, tab_vmem, sem).start()
        pltpu.make_async_copy(tab_hbm, tab_vmem, sem).wait()

    def outer(o, carry):
        base = o * U
        for u8 in range(U // 8):
            rows = []
            for v in range(8):
                mi = base + u8 * 8 + v
                rows.append(tab_vmem[ridx_ref[0, 0, mi]])
            chunk = jnp.concatenate(rows, axis=0)
            off = pl.multiple_of(base + u8 * 8, 8)
            o_ref[pl.ds(off, 8), :] = chunk
        return carry

    jax.lax.fori_loop(0, GB // U, outer, 0)


def _gather(tab, ridx_col):
    return pl.pallas_call(
        _gather_kernel,
        grid=(NBLK,),
        in_specs=[
            pl.BlockSpec(memory_space=pl.ANY),
            pl.BlockSpec((1, 1, GB), lambda i: (i, 0, 0),
                         memory_space=pltpu.SMEM),
        ],
        out_specs=pl.BlockSpec((GB, 128), lambda i: (i, 0)),
        out_shape=jax.ShapeDtypeStruct((NP, 128), jnp.float32),
        scratch_shapes=[pltpu.VMEM((NPH, 1, 128), jnp.float32),
                        pltpu.SemaphoreType.DMA],
        compiler_params=pltpu.CompilerParams(
            dimension_semantics=("arbitrary",),
            vmem_limit_bytes=50 * 1024 * 1024),
        name="gather_rows",
    )(tab, ridx_col)


def _combine_kernel(g0, g1, g2, g3, g4, g5, g6, bits_ref, o_ref):
    i = pl.program_id(0)
    bits = bits_ref[...]
    acc = g0[...] * ((bits & 1).astype(jnp.float32))
    for j, gr in enumerate((g1, g2, g3, g4, g5, g6), start=1):
        mj = ((bits >> j) & 1).astype(jnp.float32)
        acc = acc + gr[...] * mj
    folded = acc[:, :CH] + acc[:, CH:]
    rows = i * GB + jax.lax.broadcasted_iota(jnp.int32, (GB, CH), 0)
    o_ref[...] = jnp.where(rows < N, folded, 0.0)


def _combine(gs, bits):
    row128 = pl.BlockSpec((GB, 128), lambda i: (i, 0))
    return pl.pallas_call(
        _combine_kernel,
        grid=(NBLK,),
        in_specs=[row128] * 7 + [row128],
        out_specs=pl.BlockSpec((GB, CH), lambda i: (i, 0)),
        out_shape=jax.ShapeDtypeStruct((NP, CH), jnp.float32),
        compiler_params=pltpu.CompilerParams(
            dimension_semantics=("arbitrary",)),
        name="combine7",
    )(*gs, bits)


def kernel(x, neigh_orders, g1, b1, g2, b2, g3, b3, g4, b4,
           W1, wb1, W2, wb2, W3, wb3, W4, wb4):
    xp = jnp.pad(x, ((0, NP - N), (0, 0)))
    idx = neigh_orders.astype(jnp.int32).reshape(N, 7)
    idxp = jnp.pad(idx, ((0, NP - N), (0, 0)))
    ridx_cols = [(idxp[:, j] >> 1).reshape(NBLK, 1, GB) for j in range(7)]

    lanelt = (jnp.arange(128) < CH)[None, :]
    bits = jnp.zeros((NP, 128), jnp.int32)
    for j in range(7):
        hj = idxp[:, j:j + 1] & 1
        bj = jnp.where(lanelt, 1 - hj, hj)
        bits = bits | (bj << j)

    gb = [(g1, b1), (g2, b2), (g3, b3), (g4, b4)]
    ww = [(W1, wb1), (W2, wb2), (W3, wb3), (W4, wb4)]

    blocks = [xp]
    stats = []
    for k in range(1, 5):
        cin = CH * k
        stats.append(_stats(blocks[-1]))
        mu = jnp.concatenate([s[0:1] for s in stats], axis=1)
        var = jnp.concatenate([s[1:2] for s in stats], axis=1)
        g, b = gb[k - 1]
        W, wb = ww[k - 1]
        w2 = W.reshape(7, cin, CH).transpose(1, 0, 2).reshape(cin, 7 * CH)
        wbpad = jnp.concatenate(
            [wb.reshape(1, CH), jnp.zeros((1, 6 * CH), jnp.float32)], axis=1)
        zs = _transform(k, blocks, mu, var, g.reshape(1, cin),
                        b.reshape(1, cin), w2, wbpad)
        gs = [_gather(zs[j], ridx_cols[j]) for j in range(7)]
        blocks.append(_combine(gs, bits))
    return blocks[4][:N]


# speedup vs baseline: 2.7919x; 1.0485x over previous
import functools

import jax
import jax.numpy as jnp
from jax.experimental import pallas as pl
from jax.experimental.pallas import tpu as pltpu

N = 163842
CH = 64
EPS = 1e-5
SLOPE = 0.2

GB = 1024
NBLK = 161
NP = NBLK * GB
NPH = NP // 2
SB = 7168
U = 64


def _stats_kernel(x_ref, o_ref, acc_ref):
    i = pl.program_id(0)

    @pl.when(i == 0)
    def _():
        acc_ref[...] = jnp.zeros_like(acc_ref)

    xb = x_ref[...]
    acc_ref[0:1, :] = acc_ref[0:1, :] + jnp.sum(xb, axis=0, keepdims=True)
    acc_ref[1:2, :] = acc_ref[1:2, :] + jnp.sum(xb * xb, axis=0, keepdims=True)

    @pl.when(i == pl.num_programs(0) - 1)
    def _():
        mu = acc_ref[0:1, :] * (1.0 / N)
        var = acc_ref[1:2, :] * (1.0 / N) - mu * mu
        o_ref[0:1, :] = mu
        o_ref[1:2, :] = var


def _stats(xp):
    return pl.pallas_call(
        _stats_kernel,
        grid=(NP // SB,),
        in_specs=[pl.BlockSpec((SB, CH), lambda i: (i, 0))],
        out_specs=pl.BlockSpec((2, CH), lambda i: (0, 0)),
        out_shape=jax.ShapeDtypeStruct((2, CH), jnp.float32),
        scratch_shapes=[pltpu.VMEM((2, CH), jnp.float32)],
        compiler_params=pltpu.CompilerParams(
            dimension_semantics=("arbitrary",)),
        name="bn_stats",
    )(xp)


def _transform_kernel(k, *refs):
    x_refs = refs[:k]
    mu_ref, var_ref, g_ref, b_ref, w2_ref, wb_ref = refs[k:k + 6]
    z_refs = refs[k + 6:k + 13]
    y_ref = refs[k + 13]
    for m in range(k):
        sl = slice(m * CH, (m + 1) * CH)
        xm = x_refs[m][...]
        yv = ((xm - mu_ref[0:1, sl])
              * jax.lax.rsqrt(var_ref[0:1, sl] + EPS)
              * g_ref[0:1, sl] + b_ref[0:1, sl])
        y_ref[:, sl] = jnp.where(yv > 0, yv, SLOPE * yv)
    z = jnp.dot(y_ref[...], w2_ref[...],
                preferred_element_type=jnp.float32) + wb_ref[...]
    z3 = z.reshape(GB // 2, 2, 7 * CH)
    for j in range(7):
        sl = slice(j * CH, (j + 1) * CH)
        packed = jnp.concatenate([z3[:, 0, sl], z3[:, 1, sl]], axis=-1)
        z_refs[j][...] = packed.reshape(GB // 2, 1, 128)


def _transform(k, xs, mu, var, g, b, w2, wbpad):
    cin = CH * k
    row_spec = pl.BlockSpec((GB, CH), lambda i: (i, 0))

    def cspec(shape):
        return pl.BlockSpec(shape, lambda i: tuple(0 for _ in shape))

    return pl.pallas_call(
        functools.partial(_transform_kernel, k),
        grid=(NBLK,),
        in_specs=[row_spec] * k + [cspec((1, cin)), cspec((1, cin)),
                                   cspec((1, cin)), cspec((1, cin)),
                                   cspec((cin, 7 * CH)), cspec((1, 7 * CH))],
        out_specs=[pl.BlockSpec((GB // 2, 1, 128), lambda i: (i, 0, 0))] * 7,
        out_shape=[jax.ShapeDtypeStruct((NPH, 1, 128), jnp.float32)] * 7,
        scratch_shapes=[pltpu.VMEM((GB, cin), jnp.float32)],
        compiler_params=pltpu.CompilerParams(
            dimension_semantics=("arbitrary",)),
        name=f"transform{k}",
    )(*xs, mu, var, g, b, w2, wbpad)


def _gather_kernel(tab_hbm, ridx_ref, o_ref, tab_vmem, sem):
    i = pl.program_id(0)

    @pl.when(i == 0)
    def _():
        pltpu.make_async_copy(tab_hbm, tab_vmem, sem).start()
        pltpu.make_async_copy(tab_hbm, tab_vmem, sem).wait()

    def outer(o, carry):
        base = o * U
        for u8 in range(U // 8):
            rows = []
            for v in range(8):
                mi = base + u8 * 8 + v
                rows.append(tab_vmem[ridx_ref[0, 0, mi]])
            chunk = jnp.concatenate(rows, axis=0)
            off = pl.multiple_of(base + u8 * 8, 8)
            o_ref[pl.ds(off, 8), :] = chunk
        return carry

    jax.lax.fori_loop(0, GB // U, outer, 0)


def _gather(tab, ridx_col):
    return pl.pallas_call(
        _gather_kernel,
        grid=(NBLK,),
        in_specs=[
            pl.BlockSpec(memory_space=pl.ANY),
            pl.BlockSpec((1, 1, GB), lambda i: (i, 0, 0),
                         memory_space=pltpu.SMEM),
        ],
        out_specs=pl.BlockSpec((GB, 128), lambda i: (i, 0)),
        out_shape=jax.ShapeDtypeStruct((NP, 128), jnp.float32),
        scratch_shapes=[pltpu.VMEM((NPH, 1, 128), jnp.float32),
                        pltpu.SemaphoreType.DMA],
        compiler_params=pltpu.CompilerParams(
            dimension_semantics=("arbitrary",),
            vmem_limit_bytes=50 * 1024 * 1024),
        name="gather_rows",
    )(tab, ridx_col)


def _combine_kernel(g0, g1, g2, g3, g4, g5, g6, bits_ref, o_ref):
    i = pl.program_id(0)
    bits = bits_ref[...]
    acc = g0[...] * ((bits & 1).astype(jnp.float32))
    for j, gr in enumerate((g1, g2, g3, g4, g5, g6), start=1):
        mj = ((bits >> j) & 1).astype(jnp.float32)
        acc = acc + gr[...] * mj
    folded = acc[:, :CH] + acc[:, CH:]
    rows = i * GB + jax.lax.broadcasted_iota(jnp.int32, (GB, CH), 0)
    o_ref[...] = jnp.where(rows < N, folded, 0.0)


def _combine(gs, bits):
    row128 = pl.BlockSpec((GB, 128), lambda i: (i, 0))
    return pl.pallas_call(
        _combine_kernel,
        grid=(NBLK,),
        in_specs=[row128] * 7 + [row128],
        out_specs=pl.BlockSpec((GB, CH), lambda i: (i, 0)),
        out_shape=jax.ShapeDtypeStruct((NP, CH), jnp.float32),
        compiler_params=pltpu.CompilerParams(
            dimension_semantics=("arbitrary",)),
        name="combine7",
    )(*gs, bits)


def kernel(x, neigh_orders, g1, b1, g2, b2, g3, b3, g4, b4,
           W1, wb1, W2, wb2, W3, wb3, W4, wb4):
    xp = jnp.pad(x, ((0, NP - N), (0, 0)))
    idx = neigh_orders.astype(jnp.int32).reshape(N, 7)
    idxp = jnp.pad(idx, ((0, NP - N), (0, 0)))
    ridx_cols = [(idxp[:, j] >> 1).reshape(NBLK, 1, GB) for j in range(7)]

    hrow = jnp.zeros((NP, 1), jnp.int32)
    for j in range(7):
        hrow = hrow | ((idxp[:, j:j + 1] & 1) << j)
    lanelt = (jnp.arange(128) < CH)[None, :]
    bits = jnp.where(lanelt, hrow ^ 127, hrow)

    gb = [(g1, b1), (g2, b2), (g3, b3), (g4, b4)]
    ww = [(W1, wb1), (W2, wb2), (W3, wb3), (W4, wb4)]

    blocks = [xp]
    stats = []
    for k in range(1, 5):
        cin = CH * k
        stats.append(_stats(blocks[-1]))
        mu = jnp.concatenate([s[0:1] for s in stats], axis=1)
        var = jnp.concatenate([s[1:2] for s in stats], axis=1)
        g, b = gb[k - 1]
        W, wb = ww[k - 1]
        w2 = W.reshape(7, cin, CH).transpose(1, 0, 2).reshape(cin, 7 * CH)
        wbpad = jnp.concatenate(
            [wb.reshape(1, CH), jnp.zeros((1, 6 * CH), jnp.float32)], axis=1)
        zs = _transform(k, blocks, mu, var, g.reshape(1, cin),
                        b.reshape(1, cin), w2, wbpad)
        gs = [_gather(zs[j], ridx_cols[j]) for j in range(7)]
        blocks.append(_combine(gs, bits))
    return blocks[4][:N]
